# Initial kernel scaffold; baseline (speedup 1.0000x reference)
import jax
import jax.numpy as jnp
from jax.experimental import pallas as pl


def kernel(x, edge_index, adj_values, W):
    raise NotImplementedError("write your pallas kernel here")


# speedup vs baseline: 4.5035x; 4.5035x over previous
import functools

import jax
import jax.numpy as jnp
from jax import lax
from jax.experimental import pallas as pl
from jax.experimental.pallas import tpu as pltpu
from jax.experimental.pallas import tpu_sc as plsc

N = 10000
E = 320000
D = 128

NC = 2
NS = 16
NW = NC * NS
EPT = E // NW
K = 80
NCHUNKS = EPT // K
RPT = 624
ZR = 16


def _sc_aggregate(x, src, dst, adj):
    mesh = plsc.VectorSubcoreMesh(core_axis_name="c", subcore_axis_name="s")

    @functools.partial(
        pl.kernel,
        out_type=jax.ShapeDtypeStruct((NC, N, D), jnp.float32),
        mesh=mesh,
        scratch_types=[
            pltpu.VMEM_SHARED((N, D), jnp.float32),
            pltpu.VMEM((K,), jnp.int32),
            pltpu.VMEM((K,), jnp.int32),
            pltpu.VMEM((K,), jnp.float32),
            pltpu.VMEM((K, D), jnp.float32),
            pltpu.VMEM((ZR, D), jnp.float32),
            pltpu.SemaphoreType.DMA,
        ],
    )
    def agg(x_hbm, src_hbm, dst_hbm, adj_hbm, out_hbm,
            acc, isrc, idst, av, rows, zbuf, sem):
        cid = lax.axis_index("c")
        sid = lax.axis_index("s")
        wid = cid * NS + sid

        zv = jnp.zeros((16,), jnp.float32)

        def zrow(i, _):
            for j in range(D // 16):
                zbuf[i, pl.ds(j * 16, 16)] = zv
            return ()

        lax.fori_loop(0, ZR, zrow, ())

        def zcopy(i, _):
            pltpu.sync_copy(zbuf, acc.at[pl.ds(sid * RPT + i * ZR, ZR)])
            return ()

        lax.fori_loop(0, RPT // ZR, zcopy, ())

        @pl.when(sid == NS - 1)
        def _zero_tail():
            pltpu.sync_copy(zbuf, acc.at[pl.ds(NS * RPT, ZR)])

        plsc.subcore_barrier()

        base0 = wid * EPT

        def chunk(c, _):
            base = base0 + c * K
            pltpu.sync_copy(src_hbm.at[pl.ds(base, K)], isrc)
            pltpu.sync_copy(dst_hbm.at[pl.ds(base, K)], idst)
            pltpu.sync_copy(adj_hbm.at[pl.ds(base, K)], av)
            pltpu.async_copy(x_hbm.at[isrc], rows, sem).wait()

            def group(t, _):
                a16 = av[pl.ds(t * 16, 16)]

                def edge(i, _):
                    a = a16.at[jnp.full((16,), i, jnp.int32)].get(
                        mode="promise_in_bounds")
                    k = t * 16 + i
                    for j in range(D // 16):
                        sl = pl.ds(j * 16, 16)
                        rows[k, sl] = rows[k, sl] * a
                    return ()

                lax.fori_loop(0, 16, edge, ())
                return ()

            lax.fori_loop(0, K // 16, group, ())
            pltpu.sync_copy(rows, acc.at[idst], add=True)
            return ()

        lax.fori_loop(0, NCHUNKS, chunk, ())
        plsc.subcore_barrier()

        pltpu.sync_copy(acc.at[pl.ds(sid * RPT, RPT)],
                        out_hbm.at[cid, pl.ds(sid * RPT, RPT)])

        @pl.when(sid == NS - 1)
        def _copy_tail():
            pltpu.sync_copy(acc.at[pl.ds(NS * RPT, N - NS * RPT)],
                            out_hbm.at[cid, pl.ds(NS * RPT, N - NS * RPT)])

    return agg(x, src, dst, adj)


def _tc_body(p_ref, w_ref, o_ref):
    s = p_ref[0] + p_ref[1]
    h = lax.dot_general(s, w_ref[...], (((1,), (1,)), ((), ())),
                        preferred_element_type=jnp.float32,
                        precision=lax.Precision.HIGHEST)
    o_ref[...] = jnp.maximum(h, 0.0)


def _tc_combine_matmul_relu(partials, W):
    bm = 1000
    return pl.pallas_call(
        _tc_body,
        grid=(N // bm,),
        in_specs=[
            pl.BlockSpec((NC, bm, D), lambda i: (0, i, 0)),
            pl.BlockSpec((D, D), lambda i: (0, 0)),
        ],
        out_specs=pl.BlockSpec((bm, D), lambda i: (i, 0)),
        out_shape=jax.ShapeDtypeStruct((N, D), jnp.float32),
    )(partials, W)


def kernel(x, edge_index, adj_values, W):
    dst = edge_index[0]
    src = edge_index[1]
    partials = _sc_aggregate(x, src, dst, adj_values)
    return _tc_combine_matmul_relu(partials, W)


# speedup vs baseline: 9.1001x; 2.0207x over previous
import functools

import jax
import jax.numpy as jnp
from jax import lax
from jax.experimental import pallas as pl
from jax.experimental.pallas import tpu as pltpu
from jax.experimental.pallas import tpu_sc as plsc

N = 10000
E = 320000
D = 128

NC = 2
NS = 16
NW = NC * NS
EPT = E // NW
K = 80
NCHUNKS = EPT // K
NPAIRS = (NCHUNKS - 1) // 2
RPT = 624
ZR = 16


def _sc_aggregate(x, src, dst, adj):
    mesh = plsc.VectorSubcoreMesh(core_axis_name="c", subcore_axis_name="s")

    @functools.partial(
        pl.kernel,
        out_type=jax.ShapeDtypeStruct((NC, N, D), jnp.float32),
        mesh=mesh,
        scratch_types=[
            pltpu.VMEM_SHARED((N, D), jnp.float32),
            pltpu.VMEM((K,), jnp.int32),
            pltpu.VMEM((K,), jnp.int32),
            pltpu.VMEM((K,), jnp.int32),
            pltpu.VMEM((K,), jnp.int32),
            pltpu.VMEM((K,), jnp.float32),
            pltpu.VMEM((K,), jnp.float32),
            pltpu.VMEM((K, D), jnp.float32),
            pltpu.VMEM((K, D), jnp.float32),
            pltpu.VMEM((ZR, D), jnp.float32),
            pltpu.SemaphoreType.DMA,
            pltpu.SemaphoreType.DMA,
            pltpu.SemaphoreType.DMA,
            pltpu.SemaphoreType.DMA,
        ],
    )
    def agg(x_hbm, src_hbm, dst_hbm, adj_hbm, out_hbm,
            acc, isrc0, isrc1, idst0, idst1, av0, av1, rows0, rows1,
            zbuf, isem0, isem1, gsem0, gsem1):
        cid = lax.axis_index("c")
        sid = lax.axis_index("s")
        wid = cid * NS + sid
        base0 = wid * EPT

        isrc = (isrc0, isrc1)
        idst = (idst0, idst1)
        av = (av0, av1)
        rows = (rows0, rows1)
        isem = (isem0, isem1)
        gsem = (gsem0, gsem1)

        zv = jnp.zeros((16,), jnp.float32)

        def zrow(i, _):
            for j in range(D // 16):
                zbuf[i, pl.ds(j * 16, 16)] = zv
            return ()

        lax.fori_loop(0, ZR, zrow, ())

        def zcopy(i, _):
            pltpu.sync_copy(zbuf, acc.at[pl.ds(sid * RPT + i * ZR, ZR)])
            return ()

        lax.fori_loop(0, RPT // ZR, zcopy, ())

        @pl.when(sid == NS - 1)
        def _zero_tail():
            pltpu.sync_copy(zbuf, acc.at[pl.ds(NS * RPT, ZR)])

        plsc.subcore_barrier()

        def issue_idx(c, b):
            base = jnp.minimum(base0 + c * K, E - K)
            pltpu.async_copy(src_hbm.at[pl.ds(base, K)], isrc[b], isem[b])
            pltpu.async_copy(dst_hbm.at[pl.ds(base, K)], idst[b], isem[b])
            pltpu.async_copy(adj_hbm.at[pl.ds(base, K)], av[b], isem[b])

        def wait_idx(b):
            pltpu.make_async_copy(src_hbm.at[pl.ds(0, K)], isrc[b], isem[b]).wait()
            pltpu.make_async_copy(dst_hbm.at[pl.ds(0, K)], idst[b], isem[b]).wait()
            pltpu.make_async_copy(adj_hbm.at[pl.ds(0, K)], av[b], isem[b]).wait()

        def issue_gather(b):
            pltpu.async_copy(x_hbm.at[isrc[b]], rows[b], gsem[b])

        def wait_gather(b):
            pltpu.make_async_copy(x_hbm.at[isrc[b]], rows[b], gsem[b]).wait()

        def compute(b):
            rb, ab = rows[b], av[b]

            def group(t, _):
                a16 = ab[pl.ds(t * 16, 16)]
                for i in range(16):
                    a = jnp.broadcast_to(a16[i], (16,))
                    k = t * 16 + i
                    for j in range(D // 16):
                        sl = pl.ds(j * 16, 16)
                        rb[k, sl] = rb[k, sl] * a
                return ()

            lax.fori_loop(0, K // 16, group, ())

        def scatter(b):
            pltpu.sync_copy(rows[b], acc.at[idst[b]], add=True)

        def halfstep(c, b):
            wait_gather(b)
            wait_idx(1 - b)
            issue_gather(1 - b)
            compute(b)
            scatter(b)
            issue_idx(c + 2, b)

        issue_idx(0, 0)
        wait_idx(0)
        issue_gather(0)
        issue_idx(1, 1)

        def pair(p, _):
            c0 = 2 * p
            halfstep(c0, 0)
            halfstep(c0 + 1, 1)
            return ()

        lax.fori_loop(0, NPAIRS, pair, ())

        wait_gather(0)
        compute(0)
        scatter(0)
        wait_idx(1)
        plsc.subcore_barrier()

        pltpu.sync_copy(acc.at[pl.ds(sid * RPT, RPT)],
                        out_hbm.at[cid, pl.ds(sid * RPT, RPT)])

        @pl.when(sid == NS - 1)
        def _copy_tail():
            pltpu.sync_copy(acc.at[pl.ds(NS * RPT, N - NS * RPT)],
                            out_hbm.at[cid, pl.ds(NS * RPT, N - NS * RPT)])

    return agg(x, src, dst, adj)


def _tc_body(p_ref, w_ref, o_ref):
    s = p_ref[0] + p_ref[1]
    h = lax.dot_general(s, w_ref[...], (((1,), (1,)), ((), ())),
                        preferred_element_type=jnp.float32,
                        precision=lax.Precision.HIGHEST)
    o_ref[...] = jnp.maximum(h, 0.0)


def _tc_combine_matmul_relu(partials, W):
    bm = 1000
    return pl.pallas_call(
        _tc_body,
        grid=(N // bm,),
        in_specs=[
            pl.BlockSpec((NC, bm, D), lambda i: (0, i, 0)),
            pl.BlockSpec((D, D), lambda i: (0, 0)),
        ],
        out_specs=pl.BlockSpec((bm, D), lambda i: (i, 0)),
        out_shape=jax.ShapeDtypeStruct((N, D), jnp.float32),
    )(partials, W)


def kernel(x, edge_index, adj_values, W):
    dst = edge_index[0]
    src = edge_index[1]
    partials = _sc_aggregate(x, src, dst, adj_values)
    return _tc_combine_matmul_relu(partials, W)


# speedup vs baseline: 10.0970x; 1.1095x over previous
import functools

import jax
import jax.numpy as jnp
from jax import lax
from jax.experimental import pallas as pl
from jax.experimental.pallas import tpu as pltpu
from jax.experimental.pallas import tpu_sc as plsc

N = 10000
E = 320000
D = 128

NC = 2
NS = 16
NW = NC * NS
EPT = E // NW
K = 80
NCHUNKS = EPT // K
NTRIPLES = NCHUNKS // 3
RPT = 624
ZR = 16


def _sc_aggregate(x, src, dst, adj):
    mesh = plsc.VectorSubcoreMesh(core_axis_name="c", subcore_axis_name="s")

    @functools.partial(
        pl.kernel,
        out_type=jax.ShapeDtypeStruct((NC, N, D), jnp.float32),
        mesh=mesh,
        scratch_types=[
            pltpu.VMEM_SHARED((N, D), jnp.float32),
            pltpu.VMEM((K,), jnp.int32),
            pltpu.VMEM((K,), jnp.int32),
            pltpu.VMEM((K,), jnp.int32),
            pltpu.VMEM((K,), jnp.int32),
            pltpu.VMEM((K,), jnp.int32),
            pltpu.VMEM((K,), jnp.int32),
            pltpu.VMEM((K,), jnp.float32),
            pltpu.VMEM((K,), jnp.float32),
            pltpu.VMEM((K,), jnp.float32),
            pltpu.VMEM((K, D), jnp.float32),
            pltpu.VMEM((K, D), jnp.float32),
            pltpu.VMEM((K, D), jnp.float32),
            pltpu.VMEM((ZR, D), jnp.float32),
            pltpu.SemaphoreType.DMA,
            pltpu.SemaphoreType.DMA,
            pltpu.SemaphoreType.DMA,
            pltpu.SemaphoreType.DMA,
            pltpu.SemaphoreType.DMA,
            pltpu.SemaphoreType.DMA,
            pltpu.SemaphoreType.DMA,
            pltpu.SemaphoreType.DMA,
            pltpu.SemaphoreType.DMA,
            pltpu.SemaphoreType.DMA,
            pltpu.SemaphoreType.DMA,
            pltpu.SemaphoreType.DMA,
        ],
    )
    def agg(x_hbm, src_hbm, dst_hbm, adj_hbm, out_hbm,
            acc, isrc0, isrc1, isrc2, idst0, idst1, idst2, av0, av1, av2,
            rows0, rows1, rows2, zbuf,
            isem0, isem1, isem2, dsem0, dsem1, dsem2,
            gsem0, gsem1, gsem2, ssem0, ssem1, ssem2):
        cid = lax.axis_index("c")
        sid = lax.axis_index("s")
        wid = cid * NS + sid
        base0 = wid * EPT

        isrc = (isrc0, isrc1, isrc2)
        idst = (idst0, idst1, idst2)
        av = (av0, av1, av2)
        rows = (rows0, rows1, rows2)
        isem = (isem0, isem1, isem2)
        dsem = (dsem0, dsem1, dsem2)
        gsem = (gsem0, gsem1, gsem2)
        ssem = (ssem0, ssem1, ssem2)

        zv = jnp.zeros((16,), jnp.float32)

        def zrow(i, _):
            for j in range(D // 16):
                zbuf[i, pl.ds(j * 16, 16)] = zv
            return ()

        lax.fori_loop(0, ZR, zrow, ())

        def zcopy(i, _):
            pltpu.sync_copy(zbuf, acc.at[pl.ds(sid * RPT + i * ZR, ZR)])
            return ()

        lax.fori_loop(0, RPT // ZR, zcopy, ())

        @pl.when(sid == NS - 1)
        def _zero_tail():
            pltpu.sync_copy(zbuf, acc.at[pl.ds(NS * RPT, ZR)])

        plsc.subcore_barrier()

        def _base(c):
            return jnp.minimum(base0 + c * K, E - K)

        def issue_idx(c, b):
            base = _base(c)
            pltpu.async_copy(src_hbm.at[pl.ds(base, K)], isrc[b], isem[b])
            pltpu.async_copy(adj_hbm.at[pl.ds(base, K)], av[b], isem[b])

        def wait_idx(b):
            pltpu.make_async_copy(src_hbm.at[pl.ds(0, K)], isrc[b], isem[b]).wait()
            pltpu.make_async_copy(adj_hbm.at[pl.ds(0, K)], av[b], isem[b]).wait()

        def issue_idst(c, b):
            pltpu.async_copy(dst_hbm.at[pl.ds(_base(c), K)], idst[b], dsem[b])

        def wait_idst(b):
            pltpu.make_async_copy(dst_hbm.at[pl.ds(0, K)], idst[b], dsem[b]).wait()

        def issue_gather(b):
            pltpu.async_copy(x_hbm.at[isrc[b]], rows[b], gsem[b])

        def wait_gather(b):
            pltpu.make_async_copy(x_hbm.at[isrc[b]], rows[b], gsem[b]).wait()

        def issue_scatter(b):
            pltpu.async_copy(rows[b], acc.at[idst[b]], ssem[b], add=True)

        def wait_scatter(b):
            pltpu.make_async_copy(rows[b], acc.at[idst[b]], ssem[b]).wait()

        def compute(b):
            rb, ab = rows[b], av[b]

            def group(t, _):
                a16 = ab[pl.ds(t * 16, 16)]
                for i in range(16):
                    a = jnp.broadcast_to(a16[i], (16,))
                    k = t * 16 + i
                    for j in range(D // 16):
                        sl = pl.ds(j * 16, 16)
                        rb[k, sl] = rb[k, sl] * a
                return ()

            lax.fori_loop(0, K // 16, group, ())

        def step(c, b, first):
            bn = (b + 1) % 3
            wait_gather(b)
            wait_idx(bn)
            if not first:
                wait_scatter(bn)

            @pl.when(c + 1 < NCHUNKS)
            def _g():
                issue_gather(bn)

            issue_idst(c + 1, bn)
            compute(b)
            issue_idx(c + 3, b)
            wait_idst(b)
            issue_scatter(b)

        issue_idx(0, 0)
        issue_idx(1, 1)
        issue_idx(2, 2)
        issue_idst(0, 0)
        wait_idx(0)
        issue_gather(0)

        step(0, 0, True)
        step(1, 1, True)
        step(2, 2, False)

        def triple(p, _):
            c0 = 3 * p
            step(c0, 0, False)
            step(c0 + 1, 1, False)
            step(c0 + 2, 2, False)
            return ()

        lax.fori_loop(1, NTRIPLES, triple, ())

        step(NCHUNKS - 2, 0, False)
        step(NCHUNKS - 1, 1, False)
        wait_scatter(0)
        wait_scatter(1)
        wait_idx(0)
        wait_idx(1)
        wait_idst(2)
        plsc.subcore_barrier()

        pltpu.sync_copy(acc.at[pl.ds(sid * RPT, RPT)],
                        out_hbm.at[cid, pl.ds(sid * RPT, RPT)])

        @pl.when(sid == NS - 1)
        def _copy_tail():
            pltpu.sync_copy(acc.at[pl.ds(NS * RPT, N - NS * RPT)],
                            out_hbm.at[cid, pl.ds(NS * RPT, N - NS * RPT)])

    return agg(x, src, dst, adj)


def _tc_body(p_ref, w_ref, o_ref):
    s = p_ref[0] + p_ref[1]
    h = lax.dot_general(s, w_ref[...], (((1,), (1,)), ((), ())),
                        preferred_element_type=jnp.float32,
                        precision=lax.Precision.HIGHEST)
    o_ref[...] = jnp.maximum(h, 0.0)


def _tc_combine_matmul_relu(partials, W):
    bm = 1000
    return pl.pallas_call(
        _tc_body,
        grid=(N // bm,),
        in_specs=[
            pl.BlockSpec((NC, bm, D), lambda i: (0, i, 0)),
            pl.BlockSpec((D, D), lambda i: (0, 0)),
        ],
        out_specs=pl.BlockSpec((bm, D), lambda i: (i, 0)),
        out_shape=jax.ShapeDtypeStruct((N, D), jnp.float32),
    )(partials, W)


def kernel(x, edge_index, adj_values, W):
    dst = edge_index[0]
    src = edge_index[1]
    partials = _sc_aggregate(x, src, dst, adj_values)
    return _tc_combine_matmul_relu(partials, W)
